# baseline (speedup 1.0000x reference)
import jax
import jax.numpy as jnp
from jax import lax
from jax.experimental import pallas as pl
from jax.experimental.pallas import tpu as pltpu
from jax.experimental.pallas import tpu_sc as plsc

D = 64
B = 4096 * 200
NC, NS = 2, 16
NW = NC * NS
B_PER_W = B // NW
CHUNK = 1024
N_CHUNKS = B_PER_W // CHUNK


def _gather_body(idx_hbm, table_hbm, out_hbm, idx_v, rows_v, sem):
    wid = lax.axis_index("s") * NC + lax.axis_index("c")
    base = wid * B_PER_W

    def step(i, carry):
        off = base + i * CHUNK
        pltpu.sync_copy(idx_hbm.at[pl.ds(off, CHUNK)], idx_v)
        pltpu.async_copy(table_hbm.at[idx_v], rows_v, sem).wait()
        pltpu.sync_copy(rows_v, out_hbm.at[pl.ds(off, CHUNK)])
        return carry

    lax.fori_loop(0, N_CHUNKS, step, 0)


_gather = pl.kernel(
    _gather_body,
    out_type=jax.ShapeDtypeStruct((B, D), jnp.float32),
    mesh=plsc.VectorSubcoreMesh(core_axis_name="c", subcore_axis_name="s"),
    compiler_params=pltpu.CompilerParams(use_tc_tiling_on_sc=False),
    scratch_types=[
        pltpu.VMEM((CHUNK,), jnp.int32),
        pltpu.VMEM((CHUNK, D), jnp.float32),
        pltpu.SemaphoreType.DMA,
    ],
)


def kernel(x, table):
    idx = x.reshape(-1).astype(jnp.int32)
    out = _gather(idx, table)
    return out.reshape(x.shape + (D,))


# speedup vs baseline: 1.2319x; 1.2319x over previous
import functools

import jax
import jax.numpy as jnp
from jax import lax
from jax.experimental import pallas as pl
from jax.experimental.pallas import tpu as pltpu
from jax.experimental.pallas import tpu_sc as plsc

VOCAB = 1000000
D = 64
NI, NJ = 4096, 200
B = NI * NJ
NC, NS = 2, 16
NW = NC * NS

W1 = 1024
G1 = (VOCAB + 2 * W1 - 1) // (2 * W1)
TPACK_ROWS = G1 * W1


def _repack_body(a_ref, b_ref, o_ref):
    ta = jnp.transpose(a_ref[...])
    tb = jnp.transpose(b_ref[...])
    o_ref[...] = jnp.concatenate([ta, tb], axis=1)


_repack = pl.pallas_call(
    _repack_body,
    grid=(G1,),
    in_specs=[
        pl.BlockSpec((D, W1), lambda i: (0, 2 * i)),
        pl.BlockSpec((D, W1), lambda i: (0, jnp.minimum(2 * i + 1, 975))),
    ],
    out_specs=pl.BlockSpec((W1, 128), lambda i: (i, 0)),
    out_shape=jax.ShapeDtypeStruct((TPACK_ROWS, 128), jnp.float32),
)

CHUNK = 512


def _gather_body(idx2_hbm, tpack_hbm, out_hbm, idx_v, rows_v, sem):
    nrows = idx2_hbm.shape[0]
    per_w = nrows // NW
    n_chunks = per_w // CHUNK
    wid = lax.axis_index("s") * NC + lax.axis_index("c")
    base = wid * per_w

    def step(i, carry):
        off = base + i * CHUNK
        pltpu.sync_copy(idx2_hbm.at[pl.ds(off, CHUNK)], idx_v)
        pltpu.async_copy(tpack_hbm.at[idx_v], rows_v, sem).wait()
        pltpu.sync_copy(rows_v, out_hbm.at[pl.ds(off, CHUNK)])
        return carry

    lax.fori_loop(0, n_chunks, step, 0)


def _make_gather(nrows):
    return pl.kernel(
        _gather_body,
        out_type=jax.ShapeDtypeStruct((nrows, 128), jnp.float32),
        mesh=plsc.VectorSubcoreMesh(core_axis_name="c", subcore_axis_name="s"),
        compiler_params=pltpu.CompilerParams(use_tc_tiling_on_sc=False),
        scratch_types=[
            pltpu.VMEM((CHUNK,), jnp.int32),
            pltpu.VMEM((CHUNK, 128), jnp.float32),
            pltpu.SemaphoreType.DMA,
        ],
    )


JB = 8
IB = 512
NJ_A, NJ_B = 96, 104
ROWS_A, ROWS_B = NJ_A * NI, NJ_B * NI
AOFF = NJ_A // JB


def _select_t_body_a(r_ref, x_ref, o_ref):
    blk = r_ref[...]
    h = ((x_ref[...] >> 10) & 1).astype(jnp.bool_)
    tblk = jnp.transpose(blk, (0, 2, 1))
    h3 = lax.broadcast_in_dim(h, (JB, D, IB), (0, 2))
    o_ref[...] = jnp.where(h3, tblk[:, D:, :], tblk[:, :D, :])


def _select_t_body_b(r_ref, x_ref, _prev_ref, o_ref):
    _select_t_body_a(r_ref, x_ref, o_ref)


_OUT3_TYPE = jax.ShapeDtypeStruct((NJ, D, NI), jnp.float32)

_select_a = pl.pallas_call(
    _select_t_body_a,
    grid=(NJ_A // JB, NI // IB),
    in_specs=[
        pl.BlockSpec((JB, IB, 128), lambda a, b: (a, b, 0)),
        pl.BlockSpec((JB, IB), lambda a, b: (a, b)),
    ],
    out_specs=pl.BlockSpec((JB, D, IB), lambda a, b: (a, 0, b)),
    out_shape=_OUT3_TYPE,
)

_select_b = pl.pallas_call(
    _select_t_body_b,
    grid=(NJ_B // JB, NI // IB),
    in_specs=[
        pl.BlockSpec((JB, IB, 128), lambda a, b: (a, b, 0)),
        pl.BlockSpec((JB, IB), lambda a, b: (a + AOFF, b)),
        pl.BlockSpec(memory_space=pl.ANY),
    ],
    out_specs=pl.BlockSpec((JB, D, IB), lambda a, b: (a + AOFF, 0, b)),
    out_shape=_OUT3_TYPE,
    input_output_aliases={2: 0},
)


def kernel(x, table):
    tT = jnp.transpose(table)
    xt = jnp.transpose(x).astype(jnp.int32)
    tpack = _repack(tT, tT)

    idx2 = (((xt >> 11) << 10) | (xt & 1023)).reshape(-1)

    idx2_a = lax.slice(idx2, (0,), (ROWS_A,))
    out2_a = _make_gather(ROWS_A)(idx2_a, tpack)
    r3_a = jnp.reshape(out2_a, (NJ_A, NI, 128))

    idx2_b = lax.slice(idx2, (ROWS_A,), (B,))
    out2_b = _make_gather(ROWS_B)(idx2_b, tpack)
    r3_b = jnp.reshape(out2_b, (NJ_B, NI, 128))

    buf_a = _select_a(r3_a, xt)
    out3 = _select_b(r3_b, xt, buf_a)
    return jnp.transpose(out3, (2, 0, 1))
